# baseline (speedup 1.0000x reference)
import functools

import jax
import jax.numpy as jnp
from jax import lax
from jax.experimental import pallas as pl
from jax.experimental.pallas import tpu as pltpu
from jax.experimental.pallas import tpu_sc as plsc

FEATURE_SIZE = 1000000
EMBED_DIM = 64
BATCH = 4096
FIELDS = 26

NUM_CORES = 2
NUM_SUBCORES = 16
NUM_WORKERS = NUM_CORES * NUM_SUBCORES

TOTAL = BATCH * FIELDS
PER_WORKER = TOTAL // NUM_WORKERS
CHUNK = 128
N_CHUNKS = PER_WORKER // CHUNK
OUT_ROWS = CHUNK // 2

NBUF = 2
OBUF = 2


def _mesh():
    return plsc.VectorSubcoreMesh(
        core_axis_name="c", subcore_axis_name="s",
        num_cores=NUM_CORES, num_subcores=NUM_SUBCORES)


def _body(pidx_hbm, hoff_hbm, table2_hbm, out2_hbm,
          pidx_v, hoff_v, pair_v, out_v, *sems):
    gsems = sems[:NBUF]
    osems = sems[NBUF:]
    wid = lax.axis_index("s") * NUM_CORES + lax.axis_index("c")
    base = wid * PER_WORKER
    obase = wid * (PER_WORKER // 2)
    pltpu.sync_copy(pidx_hbm.at[pl.ds(base, PER_WORKER)], pidx_v)
    pltpu.sync_copy(hoff_hbm.at[pl.ds(base, PER_WORKER)], hoff_v)

    def select_chunk(d, pb, ob):
        pair = pair_v.at[pb]
        outb = out_v.at[ob]

        def grp_body(g, carry):
            hv = hoff_v[pl.ds(d * CHUNK + 16 * g, 16)]
            for p in range(8):
                q = 8 * g + p
                off0 = hv[2 * p]
                off1 = hv[2 * p + 1]
                for j in range(4):
                    outb[q, pl.ds(16 * j, 16)] = pair[2 * q, pl.ds(off0 + 16 * j, 16)]
                    outb[q, pl.ds(64 + 16 * j, 16)] = pair[2 * q + 1, pl.ds(off1 + 16 * j, 16)]
            return carry

        lax.fori_loop(0, CHUNK // 16, grp_body, 0)

    def gather_desc(c, b):
        return pltpu.make_async_copy(
            table2_hbm.at[pidx_v.at[pl.ds(c * CHUNK, CHUNK)]],
            pair_v.at[b], gsems[b])

    def out_desc(c, ob):
        return pltpu.make_async_copy(
            out_v.at[ob],
            out2_hbm.at[pl.ds(obase + c * OUT_ROWS, OUT_ROWS)],
            osems[ob])

    gather_desc(0, 0).start()
    gather_desc(1, 1).start()

    def step(s, carry):
        for k in range(2):
            c = 2 * s + k
            gather_desc(c, k).wait()

            @pl.when(s >= 1)
            def _():
                out_desc(c - OBUF, k).wait()

            select_chunk(c, k, k)
            out_desc(c, k).start()

            @pl.when(s < N_CHUNKS // 2 - 1)
            def _():
                gather_desc(c + 2, k).start()
        return carry

    lax.fori_loop(0, N_CHUNKS // 2, step, 0)
    out_desc(N_CHUNKS - 2, 0).wait()
    out_desc(N_CHUNKS - 1, 1).wait()


@jax.jit
def _embed(pidx, hoff, table2):
    call = pl.kernel(
        _body,
        out_type=jax.ShapeDtypeStruct((TOTAL // 2, 128), jnp.float32),
        mesh=_mesh(),
        scratch_types=[
            pltpu.VMEM((PER_WORKER,), jnp.int32),
            pltpu.VMEM((PER_WORKER,), jnp.int32),
            pltpu.VMEM((NBUF, CHUNK, 128), jnp.float32),
            pltpu.VMEM((OBUF, OUT_ROWS, 128), jnp.float32),
        ] + [pltpu.SemaphoreType.DMA] * (NBUF + OBUF),
    )
    return call(pidx, hoff, table2)


def kernel(inputs, table):
    flat_idx = inputs.reshape(-1).astype(jnp.int32)
    pidx = flat_idx >> 1
    hoff = (flat_idx & 1) * EMBED_DIM
    table2 = table.reshape(FEATURE_SIZE // 2, 128)
    out2 = _embed(pidx, hoff, table2)
    return out2.reshape(BATCH, FIELDS, EMBED_DIM)


# speedup vs baseline: 1.1002x; 1.1002x over previous
import functools

import jax
import jax.numpy as jnp
from jax import lax
from jax.experimental import pallas as pl
from jax.experimental.pallas import tpu as pltpu
from jax.experimental.pallas import tpu_sc as plsc

FEATURE_SIZE = 1000000
EMBED_DIM = 64
BATCH = 4096
FIELDS = 26

NUM_CORES = 2
NUM_SUBCORES = 16
NUM_WORKERS = NUM_CORES * NUM_SUBCORES

TOTAL = BATCH * FIELDS
PER_WORKER = TOTAL // NUM_WORKERS
CHUNK = 128
N_CHUNKS = PER_WORKER // CHUNK
OUT_ROWS = CHUNK // 2

NBUF = 2
OBUF = 2


def _mesh():
    return plsc.VectorSubcoreMesh(
        core_axis_name="c", subcore_axis_name="s",
        num_cores=NUM_CORES, num_subcores=NUM_SUBCORES)


def _body(pidx_hbm, table2_hbm, out2_hbm,
          pidx_v, pair_v, out_v, *sems):
    gsems = sems[:NBUF]
    osems = sems[NBUF:]
    wid = lax.axis_index("s") * NUM_CORES + lax.axis_index("c")
    base = wid * PER_WORKER
    obase = wid * (PER_WORKER // 2)
    pltpu.sync_copy(pidx_hbm.at[pl.ds(base, PER_WORKER)], pidx_v)

    def select_chunk(d, pb, ob):
        pair = pair_v.at[pb]
        outb = out_v.at[ob]

        def grp_body(g, carry):
            for p in range(8):
                q = 8 * g + p
                for j in range(4):
                    outb[q, pl.ds(16 * j, 16)] = pair[2 * q, pl.ds(16 * j, 16)]
                    outb[q, pl.ds(64 + 16 * j, 16)] = pair[2 * q + 1, pl.ds(16 * j, 16)]
            return carry

        lax.fori_loop(0, CHUNK // 16, grp_body, 0)

    def gather_desc(c, b):
        return pltpu.make_async_copy(
            table2_hbm.at[pidx_v.at[pl.ds(c * CHUNK, CHUNK)]],
            pair_v.at[b], gsems[b])

    def out_desc(c, ob):
        return pltpu.make_async_copy(
            out_v.at[ob],
            out2_hbm.at[pl.ds(obase + c * OUT_ROWS, OUT_ROWS)],
            osems[ob])

    gather_desc(0, 0).start()
    gather_desc(1, 1).start()

    def step(s, carry):
        for k in range(2):
            c = 2 * s + k
            gather_desc(c, k).wait()

            @pl.when(s >= 1)
            def _():
                out_desc(c - OBUF, k).wait()

            select_chunk(c, k, k)
            out_desc(c, k).start()

            @pl.when(s < N_CHUNKS // 2 - 1)
            def _():
                gather_desc(c + 2, k).start()
        return carry

    lax.fori_loop(0, N_CHUNKS // 2, step, 0)
    out_desc(N_CHUNKS - 2, 0).wait()
    out_desc(N_CHUNKS - 1, 1).wait()


@jax.jit
def _embed(pidx, table2):
    call = pl.kernel(
        _body,
        out_type=jax.ShapeDtypeStruct((TOTAL // 2, 128), jnp.float32),
        mesh=_mesh(),
        scratch_types=[
            pltpu.VMEM((PER_WORKER,), jnp.int32),
            pltpu.VMEM((NBUF, CHUNK, 128), jnp.float32),
            pltpu.VMEM((OBUF, OUT_ROWS, 128), jnp.float32),
        ] + [pltpu.SemaphoreType.DMA] * (NBUF + OBUF),
    )
    return call(pidx, table2)


def kernel(inputs, table):
    flat_idx = inputs.reshape(-1).astype(jnp.int32)
    table2 = jnp.concatenate(
        [table, jnp.zeros((FEATURE_SIZE, 128 - EMBED_DIM), jnp.float32)], axis=1)
    out2 = _embed(flat_idx, table2)
    return out2.reshape(BATCH, FIELDS, EMBED_DIM)
